# baseline (speedup 1.0000x reference)
import functools

import jax
import jax.numpy as jnp
from jax import lax
from jax.experimental import pallas as pl
from jax.experimental.pallas import tpu as pltpu
from jax.experimental.pallas import tpu_sc as plsc

N = 10000
D = 128
E = 320000
K = 3

NC = 2
NS = 16
NW = NC * NS
CH = 100
EPT = E // NW
NCHUNK = EPT // CH
BLK = 20
NBLK = NCHUNK // BLK
ROWS_PT = 624
TAIL_BASE = NS * ROWS_PT
TAIL_ROWS = N - TAIL_BASE

@functools.cache
def _sc_kernels():
    mesh = plsc.VectorSubcoreMesh(
        core_axis_name="c", subcore_axis_name="s",
        num_cores=NC, num_subcores=NS)
    deg = functools.partial(
        pl.kernel,
        out_type=jax.ShapeDtypeStruct((NC, N, D), jnp.float32),
        mesh=mesh,
        scratch_types=[
            pltpu.VMEM_SHARED((N, D), jnp.float32),
            pltpu.VMEM((BLK, CH), jnp.int32),
            pltpu.VMEM((CH, D), jnp.float32),
        ],
    )(_deg_body)
    spmv = functools.partial(
        pl.kernel,
        out_type=jax.ShapeDtypeStruct((NC, N, D), jnp.float32),
        mesh=mesh,
        scratch_types=[
            pltpu.VMEM_SHARED((N, D), jnp.float32),
            pltpu.VMEM((BLK, CH), jnp.int32),
            pltpu.VMEM((BLK, CH), jnp.int32),
            pltpu.VMEM((CH, D), jnp.float32),
            pltpu.VMEM((CH, D), jnp.float32),
            pltpu.SemaphoreType.DMA,
            pltpu.SemaphoreType.DMA,
            pltpu.SemaphoreType.DMA,
            pltpu.SemaphoreType.DMA,
        ],
    )(_spmv_body)
    return deg, spmv


def _deg_body(dst_hbm, ones_hbm, zeros_hbm, out_hbm, acc, idx_v, ones_v):
    c = lax.axis_index("c")
    s = lax.axis_index("s")
    w = s * NC + c
    pltpu.sync_copy(ones_hbm, ones_v)
    base = pl.multiple_of(s * ROWS_PT, 8)
    pltpu.sync_copy(zeros_hbm.at[pl.ds(base, ROWS_PT)],
                    acc.at[pl.ds(base, ROWS_PT)])

    @pl.when(s == NS - 1)
    def _():
        pltpu.sync_copy(zeros_hbm.at[pl.ds(TAIL_BASE, TAIL_ROWS)],
                        acc.at[pl.ds(TAIL_BASE, TAIL_ROWS)])

    plsc.subcore_barrier()

    def blk_body(b, carry):
        pltpu.sync_copy(dst_hbm.at[w, b], idx_v)

        def body(j, carry2):
            pltpu.sync_copy(ones_v, acc.at[idx_v.at[j]], add=True)
            return carry2

        return lax.fori_loop(0, BLK, body, carry)

    lax.fori_loop(0, NBLK, blk_body, 0)
    plsc.subcore_barrier()
    pltpu.sync_copy(acc.at[pl.ds(base, ROWS_PT)],
                    out_hbm.at[c, pl.ds(base, ROWS_PT)])

    @pl.when(s == NS - 1)
    def _():
        pltpu.sync_copy(acc.at[pl.ds(TAIL_BASE, TAIL_ROWS)],
                        out_hbm.at[c, pl.ds(TAIL_BASE, TAIL_ROWS)])


def _spmv_body(src_hbm, dst_hbm, table_hbm, zeros_hbm, out_hbm,
               acc, sidx_v, didx_v, rows0_v, rows1_v,
               sem_g0, sem_g1, sem_s0, sem_s1):
    c = lax.axis_index("c")
    s = lax.axis_index("s")
    w = s * NC + c
    base = pl.multiple_of(s * ROWS_PT, 8)
    pltpu.sync_copy(zeros_hbm.at[pl.ds(base, ROWS_PT)],
                    acc.at[pl.ds(base, ROWS_PT)])

    @pl.when(s == NS - 1)
    def _():
        pltpu.sync_copy(zeros_hbm.at[pl.ds(TAIL_BASE, TAIL_ROWS)],
                        acc.at[pl.ds(TAIL_BASE, TAIL_ROWS)])

    plsc.subcore_barrier()

    def blk_body(b, carry):
        pltpu.sync_copy(src_hbm.at[w, b], sidx_v)
        pltpu.sync_copy(dst_hbm.at[w, b], didx_v)
        pltpu.async_copy(table_hbm.at[sidx_v.at[0]], rows0_v, sem_g0)
        pltpu.async_copy(table_hbm.at[sidx_v.at[1]], rows1_v, sem_g1)

        def body(jj, carry2):
            j = 2 * jj
            pltpu.make_async_copy(
                table_hbm.at[sidx_v.at[j]], rows0_v, sem_g0).wait()
            pltpu.async_copy(rows0_v, acc.at[didx_v.at[j]], sem_s0, add=True)
            pltpu.make_async_copy(
                table_hbm.at[sidx_v.at[j + 1]], rows1_v, sem_g1).wait()
            pltpu.async_copy(
                rows1_v, acc.at[didx_v.at[j + 1]], sem_s1, add=True)
            pltpu.make_async_copy(
                rows0_v, acc.at[didx_v.at[j]], sem_s0).wait()

            @pl.when(jj < BLK // 2 - 1)
            def _():
                pltpu.async_copy(
                    table_hbm.at[sidx_v.at[j + 2]], rows0_v, sem_g0)

            pltpu.make_async_copy(
                rows1_v, acc.at[didx_v.at[j + 1]], sem_s1).wait()

            @pl.when(jj < BLK // 2 - 1)
            def _():
                pltpu.async_copy(
                    table_hbm.at[sidx_v.at[j + 3]], rows1_v, sem_g1)

            return carry2

        return lax.fori_loop(0, BLK // 2, body, carry)

    lax.fori_loop(0, NBLK, blk_body, 0)
    plsc.subcore_barrier()
    pltpu.sync_copy(acc.at[pl.ds(base, ROWS_PT)],
                    out_hbm.at[c, pl.ds(base, ROWS_PT)])

    @pl.when(s == NS - 1)
    def _():
        pltpu.sync_copy(acc.at[pl.ds(TAIL_BASE, TAIL_ROWS)],
                        out_hbm.at[c, pl.ds(TAIL_BASE, TAIL_ROWS)])


R = 1000
G = N // R


def _dinv_of(degp_ref):
    deg = degp_ref[0] + degp_ref[1]
    return lax.rsqrt(jnp.maximum(deg, 1.0))


def _scale_body(degp_ref, x_ref, g1_ref):
    g1_ref[...] = x_ref[...] * _dinv_of(degp_ref)


_scale = pl.pallas_call(
    _scale_body,
    grid=(G,),
    in_specs=[
        pl.BlockSpec((NC, R, D), lambda i: (0, i, 0)),
        pl.BlockSpec((R, D), lambda i: (i, 0)),
    ],
    out_specs=pl.BlockSpec((R, D), lambda i: (i, 0)),
    out_shape=jax.ShapeDtypeStruct((N, D), jnp.float32),
)


def _mid_body(degp_ref, s1p_ref, tx1_ref, g2_ref):
    dinv = _dinv_of(degp_ref)
    s1 = s1p_ref[0] + s1p_ref[1]
    tx1 = -dinv * s1
    tx1_ref[...] = tx1
    g2_ref[...] = dinv * tx1


_mid = pl.pallas_call(
    _mid_body,
    grid=(G,),
    in_specs=[
        pl.BlockSpec((NC, R, D), lambda i: (0, i, 0)),
        pl.BlockSpec((NC, R, D), lambda i: (0, i, 0)),
    ],
    out_specs=[
        pl.BlockSpec((R, D), lambda i: (i, 0)),
        pl.BlockSpec((R, D), lambda i: (i, 0)),
    ],
    out_shape=[
        jax.ShapeDtypeStruct((N, D), jnp.float32),
        jax.ShapeDtypeStruct((N, D), jnp.float32),
    ],
)


def _final_body(degp_ref, x_ref, tx1_ref, s2p_ref, fc_ref, w_ref, b_ref,
                out_ref):
    dinv = _dinv_of(degp_ref)
    tx2 = (-2.0 * dinv) * (s2p_ref[0] + s2p_ref[1]) - x_ref[...]
    dot = functools.partial(jnp.dot, preferred_element_type=jnp.float32,
                            precision=lax.Precision.HIGHEST)
    acc = dot(fc_ref[0] * x_ref[...], w_ref[0])
    acc += dot(fc_ref[1] * tx1_ref[...], w_ref[1])
    acc += dot(fc_ref[2] * tx2, w_ref[2])
    out_ref[...] = acc + b_ref[...]


_final = pl.pallas_call(
    _final_body,
    grid=(G,),
    in_specs=[
        pl.BlockSpec((NC, R, D), lambda i: (0, i, 0)),
        pl.BlockSpec((R, D), lambda i: (i, 0)),
        pl.BlockSpec((R, D), lambda i: (i, 0)),
        pl.BlockSpec((NC, R, D), lambda i: (0, i, 0)),
        pl.BlockSpec((K, R, 1), lambda i: (0, i, 0)),
        pl.BlockSpec((K, D, D), lambda i: (0, 0, 0)),
        pl.BlockSpec((1, D), lambda i: (0, 0)),
    ],
    out_specs=pl.BlockSpec((R, D), lambda i: (i, 0)),
    out_shape=jax.ShapeDtypeStruct((N, D), jnp.float32),
)


def kernel(x, edge_index, filter_coeff, W, b):
    src2 = edge_index[0].reshape(NW, NBLK, BLK, CH)
    dst2 = edge_index[1].reshape(NW, NBLK, BLK, CH)
    zeros = jnp.zeros((N, D), jnp.float32)

    deg_kernel, spmv_kernel = _sc_kernels()
    degp = deg_kernel(dst2, jnp.ones((CH, D), jnp.float32), zeros)
    g1 = _scale(degp, x)
    s1p = spmv_kernel(src2, dst2, g1, zeros)
    tx1, g2 = _mid(degp, s1p)
    s2p = spmv_kernel(src2, dst2, g2, zeros)
    out = _final(degp, x, tx1, s2p, filter_coeff.reshape(K, N, 1), W,
                 b.reshape(1, D))
    return out


# speedup vs baseline: 1.0752x; 1.0752x over previous
import functools

import jax
import jax.numpy as jnp
from jax import lax
from jax.experimental import pallas as pl
from jax.experimental.pallas import tpu as pltpu
from jax.experimental.pallas import tpu_sc as plsc

N = 10000
D = 128
E = 320000
K = 3

NC = 2
NS = 16
NW = NC * NS
CH = 125
EPT = E // NW
NCHUNK = EPT // CH
BLK = 16
NBLK = NCHUNK // BLK
ROWS_PT = 624
TAIL_BASE = NS * ROWS_PT
TAIL_ROWS = N - TAIL_BASE

@functools.cache
def _sc_kernels():
    mesh = plsc.VectorSubcoreMesh(
        core_axis_name="c", subcore_axis_name="s",
        num_cores=NC, num_subcores=NS)
    deg = functools.partial(
        pl.kernel,
        out_type=jax.ShapeDtypeStruct((NC, N, D), jnp.float32),
        mesh=mesh,
        scratch_types=[
            pltpu.VMEM_SHARED((N, D), jnp.float32),
            pltpu.VMEM((BLK, CH), jnp.int32),
            pltpu.VMEM((CH, D), jnp.float32),
        ],
    )(_deg_body)
    spmv = functools.partial(
        pl.kernel,
        out_type=jax.ShapeDtypeStruct((NC, N, D), jnp.float32),
        mesh=mesh,
        scratch_types=[
            pltpu.VMEM_SHARED((N, D), jnp.float32),
            pltpu.VMEM((BLK, CH), jnp.int32),
            pltpu.VMEM((BLK, CH), jnp.int32),
            pltpu.VMEM((CH, D), jnp.float32),
            pltpu.VMEM((CH, D), jnp.float32),
            pltpu.SemaphoreType.DMA,
            pltpu.SemaphoreType.DMA,
            pltpu.SemaphoreType.DMA,
            pltpu.SemaphoreType.DMA,
        ],
    )(_spmv_body)
    return deg, spmv


def _deg_body(dst_hbm, ones_hbm, zeros_hbm, out_hbm, acc, idx_v, ones_v):
    c = lax.axis_index("c")
    s = lax.axis_index("s")
    w = s * NC + c
    pltpu.sync_copy(ones_hbm, ones_v)
    base = pl.multiple_of(s * ROWS_PT, 8)
    pltpu.sync_copy(zeros_hbm.at[pl.ds(base, ROWS_PT)],
                    acc.at[pl.ds(base, ROWS_PT)])

    @pl.when(s == NS - 1)
    def _():
        pltpu.sync_copy(zeros_hbm.at[pl.ds(TAIL_BASE, TAIL_ROWS)],
                        acc.at[pl.ds(TAIL_BASE, TAIL_ROWS)])

    plsc.subcore_barrier()

    def blk_body(b, carry):
        pltpu.sync_copy(dst_hbm.at[w, b], idx_v)

        def body(j, carry2):
            pltpu.sync_copy(ones_v, acc.at[idx_v.at[j]], add=True)
            return carry2

        return lax.fori_loop(0, BLK, body, carry)

    lax.fori_loop(0, NBLK, blk_body, 0)
    plsc.subcore_barrier()
    pltpu.sync_copy(acc.at[pl.ds(base, ROWS_PT)],
                    out_hbm.at[c, pl.ds(base, ROWS_PT)])

    @pl.when(s == NS - 1)
    def _():
        pltpu.sync_copy(acc.at[pl.ds(TAIL_BASE, TAIL_ROWS)],
                        out_hbm.at[c, pl.ds(TAIL_BASE, TAIL_ROWS)])


def _spmv_body(src_hbm, dst_hbm, table_hbm, zeros_hbm, out_hbm,
               acc, sidx_v, didx_v, rows0_v, rows1_v,
               sem_g0, sem_g1, sem_s0, sem_s1):
    c = lax.axis_index("c")
    s = lax.axis_index("s")
    w = s * NC + c
    base = pl.multiple_of(s * ROWS_PT, 8)
    pltpu.sync_copy(zeros_hbm.at[pl.ds(base, ROWS_PT)],
                    acc.at[pl.ds(base, ROWS_PT)])

    @pl.when(s == NS - 1)
    def _():
        pltpu.sync_copy(zeros_hbm.at[pl.ds(TAIL_BASE, TAIL_ROWS)],
                        acc.at[pl.ds(TAIL_BASE, TAIL_ROWS)])

    plsc.subcore_barrier()

    def blk_body(b, carry):
        pltpu.sync_copy(src_hbm.at[w, b], sidx_v)
        pltpu.sync_copy(dst_hbm.at[w, b], didx_v)
        pltpu.async_copy(table_hbm.at[sidx_v.at[0]], rows0_v, sem_g0)

        def body(jj, carry2):
            j = 2 * jj
            pltpu.make_async_copy(
                table_hbm.at[sidx_v.at[j]], rows0_v, sem_g0).wait()
            pltpu.async_copy(table_hbm.at[sidx_v.at[j + 1]], rows1_v, sem_g1)
            pltpu.sync_copy(rows0_v, acc.at[didx_v.at[j]], add=True)
            pltpu.make_async_copy(
                table_hbm.at[sidx_v.at[j + 1]], rows1_v, sem_g1).wait()

            @pl.when(jj < BLK // 2 - 1)
            def _():
                pltpu.async_copy(
                    table_hbm.at[sidx_v.at[j + 2]], rows0_v, sem_g0)

            pltpu.sync_copy(rows1_v, acc.at[didx_v.at[j + 1]], add=True)
            return carry2

        return lax.fori_loop(0, BLK // 2, body, carry)

    lax.fori_loop(0, NBLK, blk_body, 0)
    plsc.subcore_barrier()
    pltpu.sync_copy(acc.at[pl.ds(base, ROWS_PT)],
                    out_hbm.at[c, pl.ds(base, ROWS_PT)])

    @pl.when(s == NS - 1)
    def _():
        pltpu.sync_copy(acc.at[pl.ds(TAIL_BASE, TAIL_ROWS)],
                        out_hbm.at[c, pl.ds(TAIL_BASE, TAIL_ROWS)])


R = 1000
G = N // R


def _dinv_of(degp_ref):
    deg = degp_ref[0] + degp_ref[1]
    return lax.rsqrt(jnp.maximum(deg, 1.0))


def _scale_body(degp_ref, x_ref, g1_ref):
    g1_ref[...] = x_ref[...] * _dinv_of(degp_ref)


_scale = pl.pallas_call(
    _scale_body,
    grid=(G,),
    in_specs=[
        pl.BlockSpec((NC, R, D), lambda i: (0, i, 0)),
        pl.BlockSpec((R, D), lambda i: (i, 0)),
    ],
    out_specs=pl.BlockSpec((R, D), lambda i: (i, 0)),
    out_shape=jax.ShapeDtypeStruct((N, D), jnp.float32),
)


def _mid_body(degp_ref, s1p_ref, tx1_ref, g2_ref):
    dinv = _dinv_of(degp_ref)
    s1 = s1p_ref[0] + s1p_ref[1]
    tx1 = -dinv * s1
    tx1_ref[...] = tx1
    g2_ref[...] = dinv * tx1


_mid = pl.pallas_call(
    _mid_body,
    grid=(G,),
    in_specs=[
        pl.BlockSpec((NC, R, D), lambda i: (0, i, 0)),
        pl.BlockSpec((NC, R, D), lambda i: (0, i, 0)),
    ],
    out_specs=[
        pl.BlockSpec((R, D), lambda i: (i, 0)),
        pl.BlockSpec((R, D), lambda i: (i, 0)),
    ],
    out_shape=[
        jax.ShapeDtypeStruct((N, D), jnp.float32),
        jax.ShapeDtypeStruct((N, D), jnp.float32),
    ],
)


def _final_body(degp_ref, x_ref, tx1_ref, s2p_ref, fc_ref, w_ref, b_ref,
                out_ref):
    dinv = _dinv_of(degp_ref)
    tx2 = (-2.0 * dinv) * (s2p_ref[0] + s2p_ref[1]) - x_ref[...]
    dot = functools.partial(jnp.dot, preferred_element_type=jnp.float32,
                            precision=lax.Precision.HIGHEST)
    acc = dot(fc_ref[0] * x_ref[...], w_ref[0])
    acc += dot(fc_ref[1] * tx1_ref[...], w_ref[1])
    acc += dot(fc_ref[2] * tx2, w_ref[2])
    out_ref[...] = acc + b_ref[...]


_final = pl.pallas_call(
    _final_body,
    grid=(G,),
    in_specs=[
        pl.BlockSpec((NC, R, D), lambda i: (0, i, 0)),
        pl.BlockSpec((R, D), lambda i: (i, 0)),
        pl.BlockSpec((R, D), lambda i: (i, 0)),
        pl.BlockSpec((NC, R, D), lambda i: (0, i, 0)),
        pl.BlockSpec((K, R, 1), lambda i: (0, i, 0)),
        pl.BlockSpec((K, D, D), lambda i: (0, 0, 0)),
        pl.BlockSpec((1, D), lambda i: (0, 0)),
    ],
    out_specs=pl.BlockSpec((R, D), lambda i: (i, 0)),
    out_shape=jax.ShapeDtypeStruct((N, D), jnp.float32),
)


def kernel(x, edge_index, filter_coeff, W, b):
    src2 = edge_index[0].reshape(NW, NBLK, BLK, CH)
    dst2 = edge_index[1].reshape(NW, NBLK, BLK, CH)
    zeros = jnp.zeros((N, D), jnp.float32)

    deg_kernel, spmv_kernel = _sc_kernels()
    degp = deg_kernel(dst2, jnp.ones((CH, D), jnp.float32), zeros)
    g1 = _scale(degp, x)
    s1p = spmv_kernel(src2, dst2, g1, zeros)
    tx1, g2 = _mid(degp, s1p)
    s2p = spmv_kernel(src2, dst2, g2, zeros)
    out = _final(degp, x, tx1, s2p, filter_coeff.reshape(K, N, 1), W,
                 b.reshape(1, D))
    return out


# speedup vs baseline: 1.1114x; 1.0337x over previous
import functools

import jax
import jax.numpy as jnp
from jax import lax
from jax.experimental import pallas as pl
from jax.experimental.pallas import tpu as pltpu
from jax.experimental.pallas import tpu_sc as plsc

N = 10000
D = 128
E = 320000
K = 3

NC = 2
NS = 16
NW = NC * NS
CH = 125
EPT = E // NW
NCHUNK = EPT // CH
BLK = 40
NBLK = NCHUNK // BLK
ROWS_PT = 624
TAIL_BASE = NS * ROWS_PT
TAIL_ROWS = N - TAIL_BASE

@functools.cache
def _sc_kernels():
    mesh = plsc.VectorSubcoreMesh(
        core_axis_name="c", subcore_axis_name="s",
        num_cores=NC, num_subcores=NS)
    deg = functools.partial(
        pl.kernel,
        out_type=jax.ShapeDtypeStruct((NC, N, D), jnp.float32),
        mesh=mesh,
        scratch_types=[
            pltpu.VMEM_SHARED((N, D), jnp.float32),
            pltpu.VMEM((BLK, CH), jnp.int32),
            pltpu.VMEM((CH, D), jnp.float32),
        ],
    )(_deg_body)
    spmv = functools.partial(
        pl.kernel,
        out_type=jax.ShapeDtypeStruct((NC, N, D), jnp.float32),
        mesh=mesh,
        scratch_types=[
            pltpu.VMEM_SHARED((N, D), jnp.float32),
            pltpu.VMEM((BLK, CH), jnp.int32),
            pltpu.VMEM((BLK, CH), jnp.int32),
            pltpu.VMEM((CH, D), jnp.float32),
            pltpu.VMEM((CH, D), jnp.float32),
            pltpu.SemaphoreType.DMA,
            pltpu.SemaphoreType.DMA,
            pltpu.SemaphoreType.DMA,
            pltpu.SemaphoreType.DMA,
        ],
    )(_spmv_body)
    return deg, spmv


def _deg_body(dst_hbm, ones_hbm, zeros_hbm, out_hbm, acc, idx_v, ones_v):
    c = lax.axis_index("c")
    s = lax.axis_index("s")
    w = s * NC + c
    pltpu.sync_copy(ones_hbm, ones_v)
    base = pl.multiple_of(s * ROWS_PT, 8)
    pltpu.sync_copy(zeros_hbm.at[pl.ds(base, ROWS_PT)],
                    acc.at[pl.ds(base, ROWS_PT)])

    @pl.when(s == NS - 1)
    def _():
        pltpu.sync_copy(zeros_hbm.at[pl.ds(TAIL_BASE, TAIL_ROWS)],
                        acc.at[pl.ds(TAIL_BASE, TAIL_ROWS)])

    plsc.subcore_barrier()

    def blk_body(b, carry):
        pltpu.sync_copy(dst_hbm.at[w, b], idx_v)

        def body(j, carry2):
            pltpu.sync_copy(ones_v, acc.at[idx_v.at[j]], add=True)
            return carry2

        return lax.fori_loop(0, BLK, body, carry)

    lax.fori_loop(0, NBLK, blk_body, 0)
    plsc.subcore_barrier()
    pltpu.sync_copy(acc.at[pl.ds(base, ROWS_PT)],
                    out_hbm.at[c, pl.ds(base, ROWS_PT)])

    @pl.when(s == NS - 1)
    def _():
        pltpu.sync_copy(acc.at[pl.ds(TAIL_BASE, TAIL_ROWS)],
                        out_hbm.at[c, pl.ds(TAIL_BASE, TAIL_ROWS)])


def _spmv_body(src_hbm, dst_hbm, table_hbm, zeros_hbm, out_hbm,
               acc, sidx_v, didx_v, rows0_v, rows1_v,
               sem_g0, sem_g1, sem_s0, sem_s1):
    c = lax.axis_index("c")
    s = lax.axis_index("s")
    w = s * NC + c
    base = pl.multiple_of(s * ROWS_PT, 8)
    pltpu.sync_copy(zeros_hbm.at[pl.ds(base, ROWS_PT)],
                    acc.at[pl.ds(base, ROWS_PT)])

    @pl.when(s == NS - 1)
    def _():
        pltpu.sync_copy(zeros_hbm.at[pl.ds(TAIL_BASE, TAIL_ROWS)],
                        acc.at[pl.ds(TAIL_BASE, TAIL_ROWS)])

    plsc.subcore_barrier()

    def blk_body(b, carry):
        pltpu.sync_copy(src_hbm.at[w, b], sidx_v)
        pltpu.sync_copy(dst_hbm.at[w, b], didx_v)
        pltpu.async_copy(table_hbm.at[sidx_v.at[0]], rows0_v, sem_g0)

        def body(jj, carry2):
            j = 2 * jj
            pltpu.make_async_copy(
                table_hbm.at[sidx_v.at[j]], rows0_v, sem_g0).wait()
            pltpu.async_copy(table_hbm.at[sidx_v.at[j + 1]], rows1_v, sem_g1)
            pltpu.sync_copy(rows0_v, acc.at[didx_v.at[j]], add=True)
            pltpu.make_async_copy(
                table_hbm.at[sidx_v.at[j + 1]], rows1_v, sem_g1).wait()

            @pl.when(jj < BLK // 2 - 1)
            def _():
                pltpu.async_copy(
                    table_hbm.at[sidx_v.at[j + 2]], rows0_v, sem_g0)

            pltpu.sync_copy(rows1_v, acc.at[didx_v.at[j + 1]], add=True)
            return carry2

        return lax.fori_loop(0, BLK // 2, body, carry)

    lax.fori_loop(0, NBLK, blk_body, 0)
    plsc.subcore_barrier()
    pltpu.sync_copy(acc.at[pl.ds(base, ROWS_PT)],
                    out_hbm.at[c, pl.ds(base, ROWS_PT)])

    @pl.when(s == NS - 1)
    def _():
        pltpu.sync_copy(acc.at[pl.ds(TAIL_BASE, TAIL_ROWS)],
                        out_hbm.at[c, pl.ds(TAIL_BASE, TAIL_ROWS)])


R = 1000
G = N // R


def _dinv_of(degp_ref):
    deg = degp_ref[0] + degp_ref[1]
    return lax.rsqrt(jnp.maximum(deg, 1.0))


def _scale_body(degp_ref, x_ref, g1_ref):
    g1_ref[...] = x_ref[...] * _dinv_of(degp_ref)


_scale = pl.pallas_call(
    _scale_body,
    grid=(G,),
    in_specs=[
        pl.BlockSpec((NC, R, D), lambda i: (0, i, 0)),
        pl.BlockSpec((R, D), lambda i: (i, 0)),
    ],
    out_specs=pl.BlockSpec((R, D), lambda i: (i, 0)),
    out_shape=jax.ShapeDtypeStruct((N, D), jnp.float32),
)


def _mid_body(degp_ref, s1p_ref, tx1_ref, g2_ref):
    dinv = _dinv_of(degp_ref)
    s1 = s1p_ref[0] + s1p_ref[1]
    tx1 = -dinv * s1
    tx1_ref[...] = tx1
    g2_ref[...] = dinv * tx1


_mid = pl.pallas_call(
    _mid_body,
    grid=(G,),
    in_specs=[
        pl.BlockSpec((NC, R, D), lambda i: (0, i, 0)),
        pl.BlockSpec((NC, R, D), lambda i: (0, i, 0)),
    ],
    out_specs=[
        pl.BlockSpec((R, D), lambda i: (i, 0)),
        pl.BlockSpec((R, D), lambda i: (i, 0)),
    ],
    out_shape=[
        jax.ShapeDtypeStruct((N, D), jnp.float32),
        jax.ShapeDtypeStruct((N, D), jnp.float32),
    ],
)


def _final_body(degp_ref, x_ref, tx1_ref, s2p_ref, fc_ref, w_ref, b_ref,
                out_ref):
    dinv = _dinv_of(degp_ref)
    tx2 = (-2.0 * dinv) * (s2p_ref[0] + s2p_ref[1]) - x_ref[...]
    dot = functools.partial(jnp.dot, preferred_element_type=jnp.float32,
                            precision=lax.Precision.HIGHEST)
    acc = dot(fc_ref[0] * x_ref[...], w_ref[0])
    acc += dot(fc_ref[1] * tx1_ref[...], w_ref[1])
    acc += dot(fc_ref[2] * tx2, w_ref[2])
    out_ref[...] = acc + b_ref[...]


_final = pl.pallas_call(
    _final_body,
    grid=(G,),
    in_specs=[
        pl.BlockSpec((NC, R, D), lambda i: (0, i, 0)),
        pl.BlockSpec((R, D), lambda i: (i, 0)),
        pl.BlockSpec((R, D), lambda i: (i, 0)),
        pl.BlockSpec((NC, R, D), lambda i: (0, i, 0)),
        pl.BlockSpec((K, R, 1), lambda i: (0, i, 0)),
        pl.BlockSpec((K, D, D), lambda i: (0, 0, 0)),
        pl.BlockSpec((1, D), lambda i: (0, 0)),
    ],
    out_specs=pl.BlockSpec((R, D), lambda i: (i, 0)),
    out_shape=jax.ShapeDtypeStruct((N, D), jnp.float32),
)


def kernel(x, edge_index, filter_coeff, W, b):
    src2 = edge_index[0].reshape(NW, NBLK, BLK, CH)
    dst2 = edge_index[1].reshape(NW, NBLK, BLK, CH)
    zeros = jnp.zeros((N, D), jnp.float32)

    deg_kernel, spmv_kernel = _sc_kernels()
    degp = deg_kernel(dst2, jnp.ones((CH, D), jnp.float32), zeros)
    g1 = _scale(degp, x)
    s1p = spmv_kernel(src2, dst2, g1, zeros)
    tx1, g2 = _mid(degp, s1p)
    s2p = spmv_kernel(src2, dst2, g2, zeros)
    out = _final(degp, x, tx1, s2p, filter_coeff.reshape(K, N, 1), W,
                 b.reshape(1, D))
    return out


# speedup vs baseline: 1.2766x; 1.1486x over previous
import functools

import jax
import jax.numpy as jnp
from jax import lax
from jax.experimental import pallas as pl
from jax.experimental.pallas import tpu as pltpu
from jax.experimental.pallas import tpu_sc as plsc

N = 10000
D = 128
E = 320000
K = 3

NC = 2
NS = 16
NW = NC * NS
CH = 125
EPT = E // NW
NCHUNK = EPT // CH
BLK = 40
NBLK = NCHUNK // BLK
ROWS_PT = 624
TAIL_BASE = NS * ROWS_PT
TAIL_ROWS = N - TAIL_BASE
HR = 128

@functools.cache
def _sc_kernels():
    mesh = plsc.VectorSubcoreMesh(
        core_axis_name="c", subcore_axis_name="s",
        num_cores=NC, num_subcores=NS)
    deg = functools.partial(
        pl.kernel,
        out_type=jax.ShapeDtypeStruct((NC, HR, D), jnp.float32),
        mesh=mesh,
        compiler_params=pltpu.CompilerParams(needs_layout_passes=False),
        scratch_types=[
            pltpu.VMEM_SHARED((NS, HR, D), jnp.float32),
            pltpu.VMEM((HR, D), jnp.float32),
            pltpu.VMEM((BLK, CH), jnp.int32),
            pltpu.VMEM((8, D), jnp.float32),
            pltpu.VMEM((8, D), jnp.float32),
        ],
    )(_deg_body)
    spmv = functools.partial(
        pl.kernel,
        out_type=jax.ShapeDtypeStruct((NC, N, D), jnp.float32),
        mesh=mesh,
        scratch_types=[
            pltpu.VMEM_SHARED((N, D), jnp.float32),
            pltpu.VMEM((BLK, CH), jnp.int32),
            pltpu.VMEM((BLK, CH), jnp.int32),
            pltpu.VMEM((CH, D), jnp.float32),
            pltpu.VMEM((CH, D), jnp.float32),
            pltpu.SemaphoreType.DMA,
            pltpu.SemaphoreType.DMA,
            pltpu.SemaphoreType.DMA,
            pltpu.SemaphoreType.DMA,
        ],
    )(_spmv_body)
    return deg, spmv


def _deg_body(dst_hbm, out_hbm, staging, hist_v, idx_v, red_v, tmp_v):
    c = lax.axis_index("c")
    s = lax.axis_index("s")
    w = s * NC + c
    zero16 = jnp.zeros((16,), jnp.float32)
    one16 = jnp.ones((16,), jnp.float32)
    tail_mask = lax.iota(jnp.int32, 16) >= (16 - (CH - (CH // 16) * 16))

    def zrow(r, carry):
        for k in range(D // 16):
            hist_v[r, pl.ds(k * 16, 16)] = zero16
        return carry

    lax.fori_loop(0, HR, zrow, 0)

    def blk_body(b, carry):
        pltpu.sync_copy(dst_hbm.at[w, b], idx_v)

        def row_body(r, carry2):
            for k in range(CH // 16 + 1):
                start = k * 16 if (k + 1) * 16 <= CH else CH - 16
                idx16 = idx_v[r, pl.ds(start, 16)]
                rr = lax.shift_right_logical(idx16, 7)
                cc = jnp.bitwise_and(idx16, 127)
                if (k + 1) * 16 <= CH:
                    plsc.addupdate_scatter(hist_v, [rr, cc], one16)
                else:
                    plsc.addupdate_scatter(hist_v, [rr, cc], one16,
                                           mask=tail_mask)
            return carry2

        return lax.fori_loop(0, BLK, row_body, carry)

    lax.fori_loop(0, NBLK, blk_body, 0)
    pltpu.sync_copy(hist_v, staging.at[s])
    plsc.subcore_barrier()

    rbase = pl.multiple_of(s * 8, 8)

    def zred(r, carry):
        for k in range(D // 16):
            red_v[r, pl.ds(k * 16, 16)] = zero16
        return carry

    lax.fori_loop(0, 8, zred, 0)

    def tred(t, carry):
        pltpu.sync_copy(staging.at[t, pl.ds(rbase, 8)], tmp_v)

        def arow(r, carry2):
            for k in range(D // 16):
                sl = pl.ds(k * 16, 16)
                red_v[r, sl] = red_v[r, sl] + tmp_v[r, sl]
            return carry2

        return lax.fori_loop(0, 8, arow, carry)

    lax.fori_loop(0, NS, tred, 0)
    pltpu.sync_copy(red_v, out_hbm.at[c, pl.ds(rbase, 8)])


def _spmv_body(src_hbm, dst_hbm, table_hbm, zeros_hbm, out_hbm,
               acc, sidx_v, didx_v, rows0_v, rows1_v,
               sem_g0, sem_g1, sem_s0, sem_s1):
    c = lax.axis_index("c")
    s = lax.axis_index("s")
    w = s * NC + c
    base = pl.multiple_of(s * ROWS_PT, 8)
    pltpu.sync_copy(zeros_hbm.at[pl.ds(base, ROWS_PT)],
                    acc.at[pl.ds(base, ROWS_PT)])

    @pl.when(s == NS - 1)
    def _():
        pltpu.sync_copy(zeros_hbm.at[pl.ds(TAIL_BASE, TAIL_ROWS)],
                        acc.at[pl.ds(TAIL_BASE, TAIL_ROWS)])

    plsc.subcore_barrier()

    def blk_body(b, carry):
        pltpu.sync_copy(src_hbm.at[w, b], sidx_v)
        pltpu.sync_copy(dst_hbm.at[w, b], didx_v)
        pltpu.async_copy(table_hbm.at[sidx_v.at[0]], rows0_v, sem_g0)

        def body(jj, carry2):
            j = 2 * jj
            pltpu.make_async_copy(
                table_hbm.at[sidx_v.at[j]], rows0_v, sem_g0).wait()
            pltpu.async_copy(table_hbm.at[sidx_v.at[j + 1]], rows1_v, sem_g1)
            pltpu.sync_copy(rows0_v, acc.at[didx_v.at[j]], add=True)
            pltpu.make_async_copy(
                table_hbm.at[sidx_v.at[j + 1]], rows1_v, sem_g1).wait()

            @pl.when(jj < BLK // 2 - 1)
            def _():
                pltpu.async_copy(
                    table_hbm.at[sidx_v.at[j + 2]], rows0_v, sem_g0)

            pltpu.sync_copy(rows1_v, acc.at[didx_v.at[j + 1]], add=True)
            return carry2

        return lax.fori_loop(0, BLK // 2, body, carry)

    lax.fori_loop(0, NBLK, blk_body, 0)
    plsc.subcore_barrier()
    pltpu.sync_copy(acc.at[pl.ds(base, ROWS_PT)],
                    out_hbm.at[c, pl.ds(base, ROWS_PT)])

    @pl.when(s == NS - 1)
    def _():
        pltpu.sync_copy(acc.at[pl.ds(TAIL_BASE, TAIL_ROWS)],
                        out_hbm.at[c, pl.ds(TAIL_BASE, TAIL_ROWS)])


R = 1000
G = N // R


def _dinv_of(degp_ref):
    deg = degp_ref[0] + degp_ref[1]
    return lax.rsqrt(jnp.maximum(deg, 1.0))


def _scale_body(degp_ref, x_ref, g1_ref):
    g1_ref[...] = x_ref[...] * _dinv_of(degp_ref)


_scale = pl.pallas_call(
    _scale_body,
    grid=(G,),
    in_specs=[
        pl.BlockSpec((NC, R, 1), lambda i: (0, i, 0)),
        pl.BlockSpec((R, D), lambda i: (i, 0)),
    ],
    out_specs=pl.BlockSpec((R, D), lambda i: (i, 0)),
    out_shape=jax.ShapeDtypeStruct((N, D), jnp.float32),
)


def _mid_body(degp_ref, s1p_ref, tx1_ref, g2_ref):
    dinv = _dinv_of(degp_ref)
    s1 = s1p_ref[0] + s1p_ref[1]
    tx1 = -dinv * s1
    tx1_ref[...] = tx1
    g2_ref[...] = dinv * tx1


_mid = pl.pallas_call(
    _mid_body,
    grid=(G,),
    in_specs=[
        pl.BlockSpec((NC, R, 1), lambda i: (0, i, 0)),
        pl.BlockSpec((NC, R, D), lambda i: (0, i, 0)),
    ],
    out_specs=[
        pl.BlockSpec((R, D), lambda i: (i, 0)),
        pl.BlockSpec((R, D), lambda i: (i, 0)),
    ],
    out_shape=[
        jax.ShapeDtypeStruct((N, D), jnp.float32),
        jax.ShapeDtypeStruct((N, D), jnp.float32),
    ],
)


def _final_body(degp_ref, x_ref, tx1_ref, s2p_ref, fc_ref, w_ref, b_ref,
                out_ref):
    dinv = _dinv_of(degp_ref)
    tx2 = (-2.0 * dinv) * (s2p_ref[0] + s2p_ref[1]) - x_ref[...]
    dot = functools.partial(jnp.dot, preferred_element_type=jnp.float32,
                            precision=lax.Precision.HIGHEST)
    acc = dot(fc_ref[0] * x_ref[...], w_ref[0])
    acc += dot(fc_ref[1] * tx1_ref[...], w_ref[1])
    acc += dot(fc_ref[2] * tx2, w_ref[2])
    out_ref[...] = acc + b_ref[...]


_final = pl.pallas_call(
    _final_body,
    grid=(G,),
    in_specs=[
        pl.BlockSpec((NC, R, 1), lambda i: (0, i, 0)),
        pl.BlockSpec((R, D), lambda i: (i, 0)),
        pl.BlockSpec((R, D), lambda i: (i, 0)),
        pl.BlockSpec((NC, R, D), lambda i: (0, i, 0)),
        pl.BlockSpec((K, R, 1), lambda i: (0, i, 0)),
        pl.BlockSpec((K, D, D), lambda i: (0, 0, 0)),
        pl.BlockSpec((1, D), lambda i: (0, 0)),
    ],
    out_specs=pl.BlockSpec((R, D), lambda i: (i, 0)),
    out_shape=jax.ShapeDtypeStruct((N, D), jnp.float32),
)


def kernel(x, edge_index, filter_coeff, W, b):
    src2 = edge_index[0].reshape(NW, NBLK, BLK, CH)
    dst2 = edge_index[1].reshape(NW, NBLK, BLK, CH)
    zeros = jnp.zeros((N, D), jnp.float32)

    deg_kernel, spmv_kernel = _sc_kernels()
    degp = deg_kernel(dst2).reshape(NC, HR * D)[:, :N].reshape(NC, N, 1)
    g1 = _scale(degp, x)
    s1p = spmv_kernel(src2, dst2, g1, zeros)
    tx1, g2 = _mid(degp, s1p)
    s2p = spmv_kernel(src2, dst2, g2, zeros)
    out = _final(degp, x, tx1, s2p, filter_coeff.reshape(K, N, 1), W,
                 b.reshape(1, D))
    return out


# speedup vs baseline: 1.3071x; 1.0239x over previous
import functools

import jax
import jax.numpy as jnp
from jax import lax
from jax.experimental import pallas as pl
from jax.experimental.pallas import tpu as pltpu
from jax.experimental.pallas import tpu_sc as plsc

N = 10000
D = 128
E = 320000
K = 3

NC = 2
NS = 16
NW = NC * NS
CH = 125
EPT = E // NW
NCHUNK = EPT // CH
BLK = 40
NBLK = NCHUNK // BLK
ROWS_PT = 624
TAIL_BASE = NS * ROWS_PT
TAIL_ROWS = N - TAIL_BASE
HR = 128

@functools.cache
def _sc_kernels():
    mesh = plsc.VectorSubcoreMesh(
        core_axis_name="c", subcore_axis_name="s",
        num_cores=NC, num_subcores=NS)
    deg = functools.partial(
        pl.kernel,
        out_type=jax.ShapeDtypeStruct((NC, HR, D), jnp.float32),
        mesh=mesh,
        compiler_params=pltpu.CompilerParams(needs_layout_passes=False),
        scratch_types=[
            pltpu.VMEM_SHARED((NS, HR, D), jnp.float32),
            pltpu.VMEM((HR, D), jnp.float32),
            pltpu.VMEM((BLK, CH), jnp.int32),
            pltpu.VMEM((8, D), jnp.float32),
            pltpu.VMEM((8, D), jnp.float32),
        ],
    )(_deg_body)
    spmv = functools.partial(
        pl.kernel,
        out_type=jax.ShapeDtypeStruct((NC, N, D), jnp.float32),
        mesh=mesh,
        scratch_types=[
            pltpu.VMEM_SHARED((N, D), jnp.float32),
            pltpu.VMEM((BLK, CH), jnp.int32),
            pltpu.VMEM((BLK, CH), jnp.int32),
            pltpu.VMEM((CH, D), jnp.float32),
            pltpu.VMEM((CH, D), jnp.float32),
            pltpu.SemaphoreType.DMA,
            pltpu.SemaphoreType.DMA,
            pltpu.SemaphoreType.DMA,
            pltpu.SemaphoreType.DMA,
        ],
    )(_spmv_body)
    return deg, spmv


def _deg_body(dst_hbm, out_hbm, staging, hist_v, idx_v, red_v, tmp_v):
    c = lax.axis_index("c")
    s = lax.axis_index("s")
    w = s * NC + c
    zero16 = jnp.zeros((16,), jnp.float32)
    one16 = jnp.ones((16,), jnp.float32)
    tail_mask = lax.iota(jnp.int32, 16) >= (16 - (CH - (CH // 16) * 16))

    def zrow(r, carry):
        for k in range(D // 16):
            hist_v[r, pl.ds(k * 16, 16)] = zero16
        return carry

    lax.fori_loop(0, HR, zrow, 0)

    def blk_body(b, carry):
        pltpu.sync_copy(dst_hbm.at[w, b], idx_v)

        def row_body(r, carry2):
            for k in range(CH // 16 + 1):
                start = k * 16 if (k + 1) * 16 <= CH else CH - 16
                idx16 = idx_v[r, pl.ds(start, 16)]
                rr = lax.shift_right_logical(idx16, 7)
                cc = jnp.bitwise_and(idx16, 127)
                if (k + 1) * 16 <= CH:
                    plsc.addupdate_scatter(hist_v, [rr, cc], one16)
                else:
                    plsc.addupdate_scatter(hist_v, [rr, cc], one16,
                                           mask=tail_mask)
            return carry2

        return lax.fori_loop(0, BLK, row_body, carry)

    lax.fori_loop(0, NBLK, blk_body, 0)
    pltpu.sync_copy(hist_v, staging.at[s])
    plsc.subcore_barrier()

    rbase = pl.multiple_of(s * 8, 8)

    def zred(r, carry):
        for k in range(D // 16):
            red_v[r, pl.ds(k * 16, 16)] = zero16
        return carry

    lax.fori_loop(0, 8, zred, 0)

    def tred(t, carry):
        pltpu.sync_copy(staging.at[t, pl.ds(rbase, 8)], tmp_v)

        def arow(r, carry2):
            for k in range(D // 16):
                sl = pl.ds(k * 16, 16)
                red_v[r, sl] = red_v[r, sl] + tmp_v[r, sl]
            return carry2

        return lax.fori_loop(0, 8, arow, carry)

    lax.fori_loop(0, NS, tred, 0)
    pltpu.sync_copy(red_v, out_hbm.at[c, pl.ds(rbase, 8)])


def _spmv_body(src_hbm, dst_hbm, table_hbm, out_hbm,
               acc, sidx_v, didx_v, rows0_v, rows1_v,
               sem_g0, sem_g1, sem_s0, sem_s1):
    c = lax.axis_index("c")
    s = lax.axis_index("s")
    w = s * NC + c
    zero16 = jnp.zeros((16,), jnp.float32)

    def zrow(r, carry):
        for k in range(D // 16):
            rows0_v[r, pl.ds(k * 16, 16)] = zero16
        return carry

    lax.fori_loop(0, CH, zrow, 0)
    base = pl.multiple_of(s * ROWS_PT, 8)
    for p in range(ROWS_PT // 104):
        pltpu.sync_copy(rows0_v.at[pl.ds(0, 104)],
                        acc.at[pl.ds(base + p * 104, 104)])

    @pl.when(s == NS - 1)
    def _():
        pltpu.sync_copy(rows0_v.at[pl.ds(0, TAIL_ROWS)],
                        acc.at[pl.ds(TAIL_BASE, TAIL_ROWS)])

    plsc.subcore_barrier()

    def blk_body(b, carry):
        pltpu.sync_copy(src_hbm.at[w, b], sidx_v)
        pltpu.sync_copy(dst_hbm.at[w, b], didx_v)
        pltpu.async_copy(table_hbm.at[sidx_v.at[0]], rows0_v, sem_g0)

        def body(jj, carry2):
            j = 2 * jj
            pltpu.make_async_copy(
                table_hbm.at[sidx_v.at[j]], rows0_v, sem_g0).wait()
            pltpu.async_copy(table_hbm.at[sidx_v.at[j + 1]], rows1_v, sem_g1)
            pltpu.sync_copy(rows0_v, acc.at[didx_v.at[j]], add=True)
            pltpu.make_async_copy(
                table_hbm.at[sidx_v.at[j + 1]], rows1_v, sem_g1).wait()

            @pl.when(jj < BLK // 2 - 1)
            def _():
                pltpu.async_copy(
                    table_hbm.at[sidx_v.at[j + 2]], rows0_v, sem_g0)

            pltpu.sync_copy(rows1_v, acc.at[didx_v.at[j + 1]], add=True)
            return carry2

        return lax.fori_loop(0, BLK // 2, body, carry)

    lax.fori_loop(0, NBLK, blk_body, 0)
    plsc.subcore_barrier()
    pltpu.sync_copy(acc.at[pl.ds(base, ROWS_PT)],
                    out_hbm.at[c, pl.ds(base, ROWS_PT)])

    @pl.when(s == NS - 1)
    def _():
        pltpu.sync_copy(acc.at[pl.ds(TAIL_BASE, TAIL_ROWS)],
                        out_hbm.at[c, pl.ds(TAIL_BASE, TAIL_ROWS)])


R = 1000
G = N // R


def _dinv_of(degp_ref):
    deg = degp_ref[0] + degp_ref[1]
    return lax.rsqrt(jnp.maximum(deg, 1.0))


def _scale_body(degp_ref, x_ref, g1_ref):
    g1_ref[...] = x_ref[...] * _dinv_of(degp_ref)


_scale = pl.pallas_call(
    _scale_body,
    grid=(G,),
    in_specs=[
        pl.BlockSpec((NC, R, 1), lambda i: (0, i, 0)),
        pl.BlockSpec((R, D), lambda i: (i, 0)),
    ],
    out_specs=pl.BlockSpec((R, D), lambda i: (i, 0)),
    out_shape=jax.ShapeDtypeStruct((N, D), jnp.float32),
)


def _mid_body(degp_ref, s1p_ref, tx1_ref, g2_ref):
    dinv = _dinv_of(degp_ref)
    s1 = s1p_ref[0] + s1p_ref[1]
    tx1 = -dinv * s1
    tx1_ref[...] = tx1
    g2_ref[...] = dinv * tx1


_mid = pl.pallas_call(
    _mid_body,
    grid=(G,),
    in_specs=[
        pl.BlockSpec((NC, R, 1), lambda i: (0, i, 0)),
        pl.BlockSpec((NC, R, D), lambda i: (0, i, 0)),
    ],
    out_specs=[
        pl.BlockSpec((R, D), lambda i: (i, 0)),
        pl.BlockSpec((R, D), lambda i: (i, 0)),
    ],
    out_shape=[
        jax.ShapeDtypeStruct((N, D), jnp.float32),
        jax.ShapeDtypeStruct((N, D), jnp.float32),
    ],
)


def _final_body(degp_ref, x_ref, tx1_ref, s2p_ref, fc_ref, w_ref, b_ref,
                out_ref):
    dinv = _dinv_of(degp_ref)
    tx2 = (-2.0 * dinv) * (s2p_ref[0] + s2p_ref[1]) - x_ref[...]
    dot = functools.partial(jnp.dot, preferred_element_type=jnp.float32,
                            precision=lax.Precision.HIGHEST)
    acc = dot(fc_ref[0] * x_ref[...], w_ref[0])
    acc += dot(fc_ref[1] * tx1_ref[...], w_ref[1])
    acc += dot(fc_ref[2] * tx2, w_ref[2])
    out_ref[...] = acc + b_ref[...]


_final = pl.pallas_call(
    _final_body,
    grid=(G,),
    in_specs=[
        pl.BlockSpec((NC, R, 1), lambda i: (0, i, 0)),
        pl.BlockSpec((R, D), lambda i: (i, 0)),
        pl.BlockSpec((R, D), lambda i: (i, 0)),
        pl.BlockSpec((NC, R, D), lambda i: (0, i, 0)),
        pl.BlockSpec((K, R, 1), lambda i: (0, i, 0)),
        pl.BlockSpec((K, D, D), lambda i: (0, 0, 0)),
        pl.BlockSpec((1, D), lambda i: (0, 0)),
    ],
    out_specs=pl.BlockSpec((R, D), lambda i: (i, 0)),
    out_shape=jax.ShapeDtypeStruct((N, D), jnp.float32),
)


def kernel(x, edge_index, filter_coeff, W, b):
    src2 = edge_index[0].reshape(NW, NBLK, BLK, CH)
    dst2 = edge_index[1].reshape(NW, NBLK, BLK, CH)

    deg_kernel, spmv_kernel = _sc_kernels()
    degp = deg_kernel(dst2).reshape(NC, HR * D)[:, :N].reshape(NC, N, 1)
    g1 = _scale(degp, x)
    s1p = spmv_kernel(src2, dst2, g1)
    tx1, g2 = _mid(degp, s1p)
    s2p = spmv_kernel(src2, dst2, g2)
    out = _final(degp, x, tx1, s2p, filter_coeff.reshape(K, N, 1), W,
                 b.reshape(1, D))
    return out
